# Initial kernel scaffold; baseline (speedup 1.0000x reference)
import jax
import jax.numpy as jnp
from jax.experimental import pallas as pl


def kernel(edge_index, edge_vals, uuHyper, iiHyper, uuHypertemp, iiHypertemp, uuHyper_wL, iiHyper_wL, uEmbeds, iEmbeds, keepRate):
    raise NotImplementedError("write your pallas kernel here")


# speedup vs baseline: 3.9122x; 3.9122x over previous
import functools

import jax
import jax.numpy as jnp
from jax import lax
from jax.experimental import pallas as pl
from jax.experimental.pallas import tpu as pltpu
from jax.experimental.pallas import tpu_sc as plsc

N_USER = 2048
N_ITEM = 2048
LATDIM = 128
N_NODES = N_USER + N_ITEM
N_EDGES = 131072

NC, NS = 2, 16
NW = NC * NS
EDGES_PER_W = N_EDGES // NW
CHUNK = 128
N_CHUNKS = EDGES_PER_W // CHUNK
ROWS_PER_SUB = N_NODES // NS


def _spmm_sc(dst, col, vals16, x, zeros):
    mesh = plsc.VectorSubcoreMesh(core_axis_name="c", subcore_axis_name="s")

    @functools.partial(
        pl.kernel,
        out_type=jax.ShapeDtypeStruct((NC, N_NODES, LATDIM), jnp.float32),
        mesh=mesh,
        scratch_types=[
            pltpu.VMEM_SHARED((N_NODES, LATDIM), jnp.float32),
            pltpu.VMEM((CHUNK,), jnp.int32),
            pltpu.VMEM((CHUNK,), jnp.int32),
            pltpu.VMEM((CHUNK, 16), jnp.float32),
            pltpu.VMEM((CHUNK, LATDIM), jnp.float32),
        ],
    )
    def k(dst_hbm, col_hbm, vals_hbm, x_hbm, z_hbm, out_hbm,
          acc, dst_v, col_v, vals_v, rows_v):
        cid = lax.axis_index("c")
        sid = lax.axis_index("s")
        wid = cid * NS + sid
        pltpu.sync_copy(z_hbm.at[pl.ds(sid * ROWS_PER_SUB, ROWS_PER_SUB)],
                        acc.at[pl.ds(sid * ROWS_PER_SUB, ROWS_PER_SUB)])
        plsc.subcore_barrier()

        @pl.loop(0, N_CHUNKS)
        def _(ci):
            base = wid * EDGES_PER_W + ci * CHUNK
            pltpu.sync_copy(dst_hbm.at[pl.ds(base, CHUNK)], dst_v)
            pltpu.sync_copy(col_hbm.at[pl.ds(base, CHUNK)], col_v)
            pltpu.sync_copy(vals_hbm.at[pl.ds(base, CHUNK)], vals_v)
            pltpu.sync_copy(x_hbm.at[col_v], rows_v)

            @pl.loop(0, CHUNK)
            def _(e):
                v = vals_v.at[pl.ds(e, 1), :][...]
                for jb in range(LATDIM // 16):
                    slc = (pl.ds(e, 1), pl.ds(jb * 16, 16))
                    rows_v.at[slc][...] = rows_v.at[slc][...] * v

            pltpu.sync_copy(rows_v, acc.at[dst_v], add=True)

        plsc.subcore_barrier()
        pltpu.sync_copy(acc.at[pl.ds(sid * ROWS_PER_SUB, ROWS_PER_SUB)],
                        out_hbm.at[cid, pl.ds(sid * ROWS_PER_SUB, ROWS_PER_SUB)])

    parts = k(dst, col, vals16, x, zeros)
    return parts[0] + parts[1]


def _mm_kernel_relu_skip(w_ref, x_ref, xs_ref, o_ref):
    acc = jnp.dot(w_ref[...], x_ref[...], preferred_element_type=jnp.float32)
    o_ref[...] = jnp.maximum(acc, 0.0) + xs_ref[...]


def _mm_kernel_relu(w_ref, x_ref, o_ref):
    acc = jnp.dot(w_ref[...], x_ref[...], preferred_element_type=jnp.float32)
    o_ref[...] = jnp.maximum(acc, 0.0)


def _mm_kernel_plain(w_ref, x_ref, o_ref):
    o_ref[...] = jnp.dot(w_ref[...], x_ref[...],
                         preferred_element_type=jnp.float32)


def _hyper_mm(w, x, mode):
    n, d = w.shape[0], x.shape[1]
    blk = 256
    in_specs = [
        pl.BlockSpec((blk, w.shape[1]), lambda i: (i, 0)),
        pl.BlockSpec((x.shape[0], d), lambda i: (0, 0)),
    ]
    if mode == "relu_skip":
        fn = _mm_kernel_relu_skip
        in_specs.append(pl.BlockSpec((blk, d), lambda i: (i, 0)))
        args = (w, x, x)
    elif mode == "relu":
        fn = _mm_kernel_relu
        args = (w, x)
    else:
        fn = _mm_kernel_plain
        args = (w, x)
    return pl.pallas_call(
        fn,
        grid=(n // blk,),
        in_specs=in_specs,
        out_specs=pl.BlockSpec((blk, d), lambda i: (i, 0)),
        out_shape=jax.ShapeDtypeStruct((n, d), jnp.float32),
    )(*args)


def _learned_kernel(hu_ref, hut_ref, wlx_ref, e_ref, o_ref):
    a = hu_ref[...] * 2.0
    b = hut_ref[...] * 2.0
    an = a / jnp.maximum(
        jnp.sqrt(jnp.sum(a * a, axis=1, keepdims=True)), 1e-8)
    bn = b / jnp.maximum(
        jnp.sqrt(jnp.sum(b * b, axis=1, keepdims=True)), 1e-8)
    t = lax.dot_general(bn, wlx_ref[...], (((0,), (0,)), ((), ())),
                        preferred_element_type=jnp.float32)
    y = jnp.dot(an, t, preferred_element_type=jnp.float32)
    o_ref[...] = jnp.maximum(y, 0.0) + e_ref[...]


def _learned(hu, hut, wlx, e):
    n, d = hu.shape
    return pl.pallas_call(
        _learned_kernel,
        out_shape=jax.ShapeDtypeStruct((n, d), jnp.float32),
    )(hu, hut, wlx, e)


def kernel(edge_index, edge_vals, uuHyper, iiHyper, uuHypertemp, iiHypertemp,
           uuHyper_wL, iiHyper_wL, uEmbeds, iEmbeds, keepRate=1):
    dst = edge_index[0]
    col = edge_index[1]
    vals16 = jnp.tile(edge_vals[:, None], (1, 16))
    embeds = jnp.concatenate([uEmbeds, iEmbeds], axis=0)
    zeros = jnp.zeros((N_NODES, LATDIM), jnp.float32)

    hU = _hyper_mm(uuHyper, uEmbeds, "relu_skip")
    hI = _hyper_mm(iiHyper, iEmbeds, "relu_skip")
    hUt = _hyper_mm(uuHypertemp, uEmbeds, "relu")
    hIt = _hyper_mm(iiHypertemp, iEmbeds, "relu")
    wlu = _hyper_mm(uuHyper_wL, uEmbeds, "plain")
    wli = _hyper_mm(iiHyper_wL, iEmbeds, "plain")

    LU = _learned(hU, hUt, wlu, uEmbeds)
    LI = _learned(hI, hIt, wli, iEmbeds)

    s1 = _spmm_sc(dst, col, vals16, embeds, zeros)
    s2 = _spmm_sc(dst, col, vals16, s1, zeros)

    embeds_out = embeds + s1 + s2
    hy = jnp.concatenate([hU, hI], axis=0)
    lh = jnp.concatenate([LU, LI], axis=0)
    return (embeds_out, s1, s2, hy, hy, lh, lh)


# speedup vs baseline: 8.3620x; 2.1374x over previous
import dataclasses
import functools

import jax
import jax.numpy as jnp
from jax import lax
from jax.experimental import pallas as pl
from jax.experimental.pallas import tpu as pltpu
from jax.experimental.pallas import tpu_sc as plsc

N_USER = 2048
N_ITEM = 2048
LATDIM = 128
N_NODES = N_USER + N_ITEM
N_EDGES = 131072

NC, NS = 2, 16
NW = NC * NS
EDGES_PER_W = N_EDGES // NW
CHUNK = 128
N_CHUNKS = EDGES_PER_W // CHUNK
ROWS_PER_SUB = N_NODES // NS


def _spmm_sc(dst, col, vals, x, zeros):
    mesh = plsc.VectorSubcoreMesh(core_axis_name="c", subcore_axis_name="s")
    cp = pltpu.CompilerParams()
    if "needs_layout_passes" in pltpu.CompilerParams.__dataclass_fields__:
        cp = dataclasses.replace(cp, needs_layout_passes=False)

    @functools.partial(
        pl.kernel,
        compiler_params=cp,
        out_type=jax.ShapeDtypeStruct((NC, N_NODES, LATDIM), jnp.float32),
        mesh=mesh,
        scratch_types=[
            pltpu.VMEM_SHARED((N_NODES, LATDIM), jnp.float32),
            pltpu.VMEM((N_CHUNKS, CHUNK), jnp.int32),
            pltpu.VMEM((N_CHUNKS, CHUNK), jnp.int32),
            pltpu.VMEM((EDGES_PER_W,), jnp.float32),
            pltpu.VMEM((CHUNK, LATDIM), jnp.float32),
            pltpu.VMEM((CHUNK, LATDIM), jnp.float32),
            pltpu.SemaphoreType.DMA,
            pltpu.SemaphoreType.DMA,
        ],
    )
    def k(dst_hbm, col_hbm, vals_hbm, x_hbm, z_hbm, out_hbm,
          acc, dst_v, col_v, vals_v, rows0, rows1, gsem0, gsem1):
        cid = lax.axis_index("c")
        sid = lax.axis_index("s")
        wid = cid * NS + sid
        stripe = pl.ds(sid * ROWS_PER_SUB, ROWS_PER_SUB)
        pltpu.sync_copy(z_hbm.at[stripe], acc.at[stripe])
        pltpu.sync_copy(dst_hbm.at[wid], dst_v)
        pltpu.sync_copy(col_hbm.at[wid], col_v)
        pltpu.sync_copy(vals_hbm.at[wid], vals_v)
        plsc.subcore_barrier()

        rows = (rows0, rows1)
        gsem = (gsem0, gsem1)

        def fire(b, ci):
            ci = jnp.minimum(ci, N_CHUNKS - 1)
            return pltpu.async_copy(x_hbm.at[col_v.at[ci]], rows[b], gsem[b])

        def drain(b, ci):
            ci = jnp.minimum(ci, N_CHUNKS - 1)
            pltpu.make_async_copy(x_hbm.at[col_v.at[ci]],
                                  rows[b], gsem[b]).wait()

        def scale_and_scatter(b, ci):
            @pl.loop(0, CHUNK)
            def _(e):
                v = plsc.load_gather(
                    vals_v, [jnp.full((16,), ci * CHUNK + e, jnp.int32)])
                for jb in range(LATDIM // 16):
                    slc = (e, pl.ds(jb * 16, 16))
                    rows[b][slc] = rows[b][slc] * v

            pltpu.sync_copy(rows[b], acc.at[dst_v.at[ci]], add=True)

        fire(0, 0)

        @pl.loop(0, N_CHUNKS, step=2)
        def _(cg):
            fire(1, cg + 1)
            drain(0, cg)
            scale_and_scatter(0, cg)
            fire(0, cg + 2)
            drain(1, cg + 1)
            scale_and_scatter(1, cg + 1)

        drain(0, N_CHUNKS)

        plsc.subcore_barrier()
        pltpu.sync_copy(acc.at[stripe], out_hbm.at[cid, stripe])

    parts = k(dst, col, vals, x, zeros)
    return parts[0] + parts[1]


def _mm_kernel_relu_skip(w_ref, x_ref, xs_ref, o_ref):
    acc = jnp.dot(w_ref[...], x_ref[...], preferred_element_type=jnp.float32)
    o_ref[...] = jnp.maximum(acc, 0.0) + xs_ref[...]


def _mm_kernel_relu(w_ref, x_ref, o_ref):
    acc = jnp.dot(w_ref[...], x_ref[...], preferred_element_type=jnp.float32)
    o_ref[...] = jnp.maximum(acc, 0.0)


def _mm_kernel_plain(w_ref, x_ref, o_ref):
    o_ref[...] = jnp.dot(w_ref[...], x_ref[...],
                         preferred_element_type=jnp.float32)


def _hyper_mm(w, x, mode):
    n, d = w.shape[0], x.shape[1]
    blk = 256
    in_specs = [
        pl.BlockSpec((blk, w.shape[1]), lambda i: (i, 0)),
        pl.BlockSpec((x.shape[0], d), lambda i: (0, 0)),
    ]
    if mode == "relu_skip":
        fn = _mm_kernel_relu_skip
        in_specs.append(pl.BlockSpec((blk, d), lambda i: (i, 0)))
        args = (w, x, x)
    elif mode == "relu":
        fn = _mm_kernel_relu
        args = (w, x)
    else:
        fn = _mm_kernel_plain
        args = (w, x)
    return pl.pallas_call(
        fn,
        grid=(n // blk,),
        in_specs=in_specs,
        out_specs=pl.BlockSpec((blk, d), lambda i: (i, 0)),
        out_shape=jax.ShapeDtypeStruct((n, d), jnp.float32),
    )(*args)


def _learned_kernel(hu_ref, hut_ref, wlx_ref, e_ref, o_ref):
    a = hu_ref[...] * 2.0
    b = hut_ref[...] * 2.0
    an = a / jnp.maximum(
        jnp.sqrt(jnp.sum(a * a, axis=1, keepdims=True)), 1e-8)
    bn = b / jnp.maximum(
        jnp.sqrt(jnp.sum(b * b, axis=1, keepdims=True)), 1e-8)
    t = lax.dot_general(bn, wlx_ref[...], (((0,), (0,)), ((), ())),
                        preferred_element_type=jnp.float32)
    y = jnp.dot(an, t, preferred_element_type=jnp.float32)
    o_ref[...] = jnp.maximum(y, 0.0) + e_ref[...]


def _learned(hu, hut, wlx, e):
    n, d = hu.shape
    return pl.pallas_call(
        _learned_kernel,
        out_shape=jax.ShapeDtypeStruct((n, d), jnp.float32),
    )(hu, hut, wlx, e)


def kernel(edge_index, edge_vals, uuHyper, iiHyper, uuHypertemp, iiHypertemp,
           uuHyper_wL, iiHyper_wL, uEmbeds, iEmbeds, keepRate=1):
    dst = edge_index[0].reshape(NW, N_CHUNKS, CHUNK)
    col = edge_index[1].reshape(NW, N_CHUNKS, CHUNK)
    vals = edge_vals.reshape(NW, EDGES_PER_W)
    embeds = jnp.concatenate([uEmbeds, iEmbeds], axis=0)
    zeros = jnp.zeros((N_NODES, LATDIM), jnp.float32)

    hU = _hyper_mm(uuHyper, uEmbeds, "relu_skip")
    hI = _hyper_mm(iiHyper, iEmbeds, "relu_skip")
    hUt = _hyper_mm(uuHypertemp, uEmbeds, "relu")
    hIt = _hyper_mm(iiHypertemp, iEmbeds, "relu")
    wlu = _hyper_mm(uuHyper_wL, uEmbeds, "plain")
    wli = _hyper_mm(iiHyper_wL, iEmbeds, "plain")

    LU = _learned(hU, hUt, wlu, uEmbeds)
    LI = _learned(hI, hIt, wli, iEmbeds)

    s1 = _spmm_sc(dst, col, vals, embeds, zeros)
    s2 = _spmm_sc(dst, col, vals, s1, zeros)

    embeds_out = embeds + s1 + s2
    hy = jnp.concatenate([hU, hI], axis=0)
    lh = jnp.concatenate([LU, LI], axis=0)
    return (embeds_out, s1, s2, hy, hy, lh, lh)
